# Initial kernel scaffold; baseline (speedup 1.0000x reference)
import jax
import jax.numpy as jnp
from jax.experimental import pallas as pl


def kernel(embedding, edge_index, edge_weight):
    raise NotImplementedError("write your pallas kernel here")


# speedup vs baseline: 3.6719x; 3.6719x over previous
import functools

import jax
import jax.numpy as jnp
from jax import lax
from jax.experimental import pallas as pl
from jax.experimental.pallas import tpu as pltpu
from jax.experimental.pallas import tpu_sc as plsc

N = 10000
E = 320000
D = 128
NC = 2
NS = 16
LANES = 16

CHUNK = 80
EDGES_PER_SUB = E // (NC * NS)
CHUNKS_PER_SUB = EDGES_PER_SUB // CHUNK
ROWS_PER_TILE = 632
N_PAD = NS * ROWS_PER_TILE


def _sc_layer_body(x_hbm, src_hbm, dst_hbm, w_hbm, out_hbm,
                   idx_v, dst_v, w_v, rows_v, acc, sem):
    c = lax.axis_index("c")
    s = lax.axis_index("s")
    base = (c * NS + s) * EDGES_PER_SUB

    def zfill(i, _):
        for g in range(D // LANES):
            rows_v[i, pl.ds(g * LANES, LANES)] = jnp.zeros((LANES,), jnp.float32)
        return 0
    lax.fori_loop(0, CHUNK, zfill, 0)
    for k in range(7):
        pltpu.sync_copy(rows_v, acc.at[pl.ds(s * ROWS_PER_TILE + k * CHUNK, CHUNK)])
    pltpu.sync_copy(rows_v.at[pl.ds(0, 72)],
                    acc.at[pl.ds(s * ROWS_PER_TILE + 7 * CHUNK, 72)])
    plsc.subcore_barrier()

    def chunk_body(t, _):
        off = base + t * CHUNK
        pltpu.sync_copy(src_hbm.at[pl.ds(off, CHUNK)], idx_v)
        pltpu.sync_copy(dst_hbm.at[pl.ds(off, CHUNK)], dst_v)
        pltpu.sync_copy(w_hbm.at[pl.ds(off, CHUNK)], w_v)
        pltpu.async_copy(x_hbm.at[idx_v], rows_v, sem).wait()

        def ebody(e16, _):
            wv = w_v[pl.ds(e16 * LANES, LANES)]
            for j in range(LANES):
                ws = jnp.take_along_axis(
                    wv, jnp.full((LANES,), j, jnp.int32), axis=0,
                    mode="promise_in_bounds")
                e = e16 * LANES + j
                for g in range(D // LANES):
                    sl = pl.ds(g * LANES, LANES)
                    rows_v[e, sl] = rows_v[e, sl] * ws
            return 0
        lax.fori_loop(0, CHUNK // LANES, ebody, 0)

        pltpu.sync_copy(rows_v, acc.at[dst_v], add=True)
        return 0

    lax.fori_loop(0, CHUNKS_PER_SUB, chunk_body, 0)
    plsc.subcore_barrier()

    @pl.when(s < NS - 1)
    def _():
        pltpu.sync_copy(acc.at[pl.ds(s * ROWS_PER_TILE, ROWS_PER_TILE)],
                        out_hbm.at[c, pl.ds(s * ROWS_PER_TILE, ROWS_PER_TILE)])

    last_base = (NS - 1) * ROWS_PER_TILE
    last_rows = N - last_base

    @pl.when(s == NS - 1)
    def _():
        pltpu.sync_copy(acc.at[pl.ds(last_base, last_rows)],
                        out_hbm.at[c, pl.ds(last_base, last_rows)])


_sc_layer = functools.partial(
    pl.kernel,
    mesh=plsc.VectorSubcoreMesh(core_axis_name="c", subcore_axis_name="s",
                                num_cores=NC),
    out_type=jax.ShapeDtypeStruct((NC, N, D), jnp.float32),
    scratch_types=[
        pltpu.VMEM((CHUNK,), jnp.int32),
        pltpu.VMEM((CHUNK,), jnp.int32),
        pltpu.VMEM((CHUNK,), jnp.float32),
        pltpu.VMEM((CHUNK, D), jnp.float32),
        pltpu.VMEM_SHARED((N_PAD, D), jnp.float32),
        pltpu.SemaphoreType.DMA,
    ],
)(_sc_layer_body)


def _combine_body(p_ref, acc_ref, x_ref, accout_ref, *, scale):
    xn = p_ref[0]
    for i in range(1, NC):
        xn = xn + p_ref[i]
    x_ref[...] = xn
    accout_ref[...] = (acc_ref[...] + xn) * scale


def _combine(partials, acc, scale):
    blk = 1000
    grid = (N // blk,)
    return pl.pallas_call(
        functools.partial(_combine_body, scale=scale),
        grid=grid,
        in_specs=[
            pl.BlockSpec((NC, blk, D), lambda i: (0, i, 0)),
            pl.BlockSpec((blk, D), lambda i: (i, 0)),
        ],
        out_specs=[
            pl.BlockSpec((blk, D), lambda i: (i, 0)),
            pl.BlockSpec((blk, D), lambda i: (i, 0)),
        ],
        out_shape=[
            jax.ShapeDtypeStruct((N, D), jnp.float32),
            jax.ShapeDtypeStruct((N, D), jnp.float32),
        ],
    )(partials, acc)


@jax.jit
def kernel(embedding, edge_index, edge_weight):
    src = edge_index[0]
    dst = edge_index[1]
    x = embedding
    acc = embedding
    for layer in range(3):
        partials = _sc_layer(x, src, dst, edge_weight)
        scale = 0.25 if layer == 2 else 1.0
        x, acc = _combine(partials, acc, scale)
    return acc
